# Initial kernel scaffold; baseline (speedup 1.0000x reference)
import jax
import jax.numpy as jnp
from jax.experimental import pallas as pl


def kernel(x, adj, W_gc, b_gc, W_lin, b_lin):
    raise NotImplementedError("write your pallas kernel here")


# speedup vs baseline: 1.1657x; 1.1657x over previous
import jax
import jax.numpy as jnp
from jax.experimental import pallas as pl
from jax.experimental.pallas import tpu as pltpu


def _support_body(x_ref, w_ref, o_ref):
    o_ref[...] = jnp.dot(x_ref[...], w_ref[...],
                         preferred_element_type=jnp.float32)


def _gc_body(adj_ref, s_ref, b_ref, w_ref, o_ref):
    h = jnp.dot(adj_ref[...], s_ref[...], preferred_element_type=jnp.float32)
    h = jnp.maximum(h + b_ref[...], 0.0)
    o_ref[...] = jnp.dot(h, w_ref[...], preferred_element_type=jnp.float32)


def _final_body(adj_ref, s_ref, bgc_ref, wlin_ref, blin_ref, o_ref):
    h = jnp.dot(adj_ref[...], s_ref[...], preferred_element_type=jnp.float32)
    h = jnp.maximum(h + bgc_ref[...], 0.0)
    logits = jax.lax.dot_general(
        h, wlin_ref[...], (((1,), (1,)), ((), ())),
        preferred_element_type=jnp.float32) + blin_ref[...]
    zmax = jnp.max(logits, axis=1, keepdims=True)
    z = logits - zmax
    lse = jnp.log(jnp.sum(jnp.exp(z), axis=1, keepdims=True))
    o_ref[...] = z - lse


def kernel(x, adj, W_gc, b_gc, W_lin, b_lin):
    n, f = x.shape
    c = W_lin.shape[0]
    tm = 400 if n % 400 == 0 else n
    grid = (n // tm,)

    b_gc2 = b_gc.reshape(1, f)
    b_lin2 = b_lin.reshape(1, c)

    params = pltpu.CompilerParams(
        dimension_semantics=("arbitrary",),
        vmem_limit_bytes=128 * 1024 * 1024,
    )

    support0 = pl.pallas_call(
        _support_body,
        grid=grid,
        in_specs=[
            pl.BlockSpec((tm, f), lambda m: (m, 0)),
            pl.BlockSpec((f, f), lambda m: (0, 0)),
        ],
        out_specs=pl.BlockSpec((tm, f), lambda m: (m, 0)),
        out_shape=jax.ShapeDtypeStruct((n, f), jnp.float32),
        compiler_params=params,
    )(x, W_gc)

    support1 = pl.pallas_call(
        _gc_body,
        grid=grid,
        in_specs=[
            pl.BlockSpec((tm, n), lambda m: (m, 0)),
            pl.BlockSpec((n, f), lambda m: (0, 0)),
            pl.BlockSpec((1, f), lambda m: (0, 0)),
            pl.BlockSpec((f, f), lambda m: (0, 0)),
        ],
        out_specs=pl.BlockSpec((tm, f), lambda m: (m, 0)),
        out_shape=jax.ShapeDtypeStruct((n, f), jnp.float32),
        compiler_params=params,
    )(adj, support0, b_gc2, W_gc)

    out = pl.pallas_call(
        _final_body,
        grid=grid,
        in_specs=[
            pl.BlockSpec((tm, n), lambda m: (m, 0)),
            pl.BlockSpec((n, f), lambda m: (0, 0)),
            pl.BlockSpec((1, f), lambda m: (0, 0)),
            pl.BlockSpec((c, f), lambda m: (0, 0)),
            pl.BlockSpec((1, c), lambda m: (0, 0)),
        ],
        out_specs=pl.BlockSpec((tm, c), lambda m: (m, 0)),
        out_shape=jax.ShapeDtypeStruct((n, c), jnp.float32),
        compiler_params=params,
    )(adj, support1, b_gc2, W_lin, b_lin2)

    return out


# speedup vs baseline: 1.2008x; 1.0301x over previous
import jax
import jax.numpy as jnp
from jax.experimental import pallas as pl
from jax.experimental.pallas import tpu as pltpu


def _support_body(x_ref, w_ref, o_ref):
    o_ref[...] = jnp.dot(x_ref[...], w_ref[...],
                         preferred_element_type=jnp.float32
                         ).astype(jnp.bfloat16)


def _gc_body(adj_ref, s_ref, b_ref, w_ref, o_ref):
    a16 = adj_ref[...].astype(jnp.bfloat16)
    h = jnp.dot(a16, s_ref[...], preferred_element_type=jnp.float32)
    h = jnp.maximum(h + b_ref[...], 0.0)
    o_ref[...] = jnp.dot(h.astype(jnp.bfloat16), w_ref[...],
                         preferred_element_type=jnp.float32
                         ).astype(jnp.bfloat16)


def _final_body(adj_ref, s_ref, bgc_ref, wlin_ref, blin_ref, o_ref):
    a16 = adj_ref[...].astype(jnp.bfloat16)
    h = jnp.dot(a16, s_ref[...], preferred_element_type=jnp.float32)
    h = jnp.maximum(h + bgc_ref[...], 0.0)
    logits = jax.lax.dot_general(
        h, wlin_ref[...], (((1,), (1,)), ((), ())),
        preferred_element_type=jnp.float32) + blin_ref[...]
    zmax = jnp.max(logits, axis=1, keepdims=True)
    z = logits - zmax
    lse = jnp.log(jnp.sum(jnp.exp(z), axis=1, keepdims=True))
    o_ref[...] = z - lse


def kernel(x, adj, W_gc, b_gc, W_lin, b_lin):
    n, f = x.shape
    c = W_lin.shape[0]
    tm = 400 if n % 400 == 0 else n
    grid = (n // tm,)

    b_gc2 = b_gc.reshape(1, f)
    b_lin2 = b_lin.reshape(1, c)
    w_gc16 = W_gc.astype(jnp.bfloat16)

    params = pltpu.CompilerParams(
        dimension_semantics=("arbitrary",),
        vmem_limit_bytes=128 * 1024 * 1024,
    )

    support0 = pl.pallas_call(
        _support_body,
        grid=grid,
        in_specs=[
            pl.BlockSpec((tm, f), lambda m: (m, 0)),
            pl.BlockSpec((f, f), lambda m: (0, 0)),
        ],
        out_specs=pl.BlockSpec((tm, f), lambda m: (m, 0)),
        out_shape=jax.ShapeDtypeStruct((n, f), jnp.bfloat16),
        compiler_params=params,
    )(x, W_gc)

    support1 = pl.pallas_call(
        _gc_body,
        grid=grid,
        in_specs=[
            pl.BlockSpec((tm, n), lambda m: (m, 0)),
            pl.BlockSpec((n, f), lambda m: (0, 0)),
            pl.BlockSpec((1, f), lambda m: (0, 0)),
            pl.BlockSpec((f, f), lambda m: (0, 0)),
        ],
        out_specs=pl.BlockSpec((tm, f), lambda m: (m, 0)),
        out_shape=jax.ShapeDtypeStruct((n, f), jnp.bfloat16),
        compiler_params=params,
    )(adj, support0, b_gc2, w_gc16)

    out = pl.pallas_call(
        _final_body,
        grid=grid,
        in_specs=[
            pl.BlockSpec((tm, n), lambda m: (m, 0)),
            pl.BlockSpec((n, f), lambda m: (0, 0)),
            pl.BlockSpec((1, f), lambda m: (0, 0)),
            pl.BlockSpec((c, f), lambda m: (0, 0)),
            pl.BlockSpec((1, c), lambda m: (0, 0)),
        ],
        out_specs=pl.BlockSpec((tm, c), lambda m: (m, 0)),
        out_shape=jax.ShapeDtypeStruct((n, c), jnp.float32),
        compiler_params=params,
    )(adj, support1, b_gc2, W_lin, b_lin2)

    return out
